# Initial kernel scaffold; baseline (speedup 1.0000x reference)
import jax
import jax.numpy as jnp
from jax.experimental import pallas as pl


def kernel(hidden_states, boundary_mask, boundary_prob, take_idx):
    raise NotImplementedError("write your pallas kernel here")


# speedup vs baseline: 13.4690x; 13.4690x over previous
import jax
import jax.numpy as jnp
from jax.experimental import pallas as pl
from jax.experimental.pallas import tpu as pltpu

LBLK = 256


def _body(mask_ref, prob_ref, tidx_ref, hid_ref, out_ref, smooth_ref, pref_ref):
    j = pl.program_id(1)
    C, D = smooth_ref.shape
    W = min(LBLK + 32, C)

    @pl.when(j == 0)
    def _scan():
        pref_ref[0] = 0
        KU = 16

        def step(cb, s):
            c0 = pl.multiple_of(cb * KU, KU)
            hblk = hid_ref[0, pl.ds(c0, KU), :]
            rows = []
            for r in range(KU):
                idx = tidx_ref[0, 0, c0 + r]
                p = jnp.clip(prob_ref[0, 0, idx], 1e-4, 1.0 - 1e-4)
                h = hblk[r:r + 1, :]
                if r == 0:
                    s = jnp.where(cb == 0, h, p * h + (1.0 - p) * s)
                else:
                    s = p * h + (1.0 - p) * s
                rows.append(s)
            smooth_ref[pl.ds(c0, KU), :] = (
                jnp.concatenate(rows, axis=0).astype(jnp.bfloat16))
            return s

        jax.lax.fori_loop(0, C // KU, step, jnp.zeros((1, D), jnp.float32))

    S = pref_ref[0]
    m = mask_ref[0]
    x = m
    k = 1
    while k < LBLK:
        x = x + jnp.concatenate([jnp.zeros((k, 1), jnp.int32), x[:-k]], axis=0)
        k *= 2
    cid = jnp.clip(S + x - 1, 0, C - 1)
    base = pl.multiple_of(jnp.clip((S - 1) // 16 * 16, 0, C - W), 16)
    off = cid - base
    oh = (off == jax.lax.broadcasted_iota(jnp.int32, (LBLK, W), 1)).astype(jnp.bfloat16)
    win = smooth_ref[pl.ds(base, W), :]
    out_ref[0] = jnp.dot(oh, win, preferred_element_type=jnp.float32)
    pref_ref[0] = S + jnp.sum(m)


def kernel(hidden_states, boundary_mask, boundary_prob, take_idx):
    B, L = boundary_mask.shape
    _, C, D = hidden_states.shape
    mask_i32 = boundary_mask.astype(jnp.int32).reshape(B, L, 1)
    prob3 = boundary_prob.reshape(B, 1, L)
    tidx3 = take_idx.reshape(B, 1, C)
    grid = (B, L // LBLK)
    return pl.pallas_call(
        _body,
        grid=grid,
        in_specs=[
            pl.BlockSpec((1, LBLK, 1), lambda b, j: (b, j, 0)),
            pl.BlockSpec((1, 1, L), lambda b, j: (b, 0, 0), memory_space=pltpu.SMEM),
            pl.BlockSpec((1, 1, C), lambda b, j: (b, 0, 0), memory_space=pltpu.SMEM),
            pl.BlockSpec((1, C, D), lambda b, j: (b, 0, 0)),
        ],
        out_specs=pl.BlockSpec((1, LBLK, D), lambda b, j: (b, j, 0)),
        out_shape=jax.ShapeDtypeStruct((B, L, D), jnp.float32),
        scratch_shapes=[
            pltpu.VMEM((C, D), jnp.bfloat16),
            pltpu.SMEM((1,), jnp.int32),
        ],
    )(mask_i32, prob3, tidx3, hidden_states)
